# Initial kernel scaffold; baseline (speedup 1.0000x reference)
import jax
import jax.numpy as jnp
from jax.experimental import pallas as pl


def kernel(x, W_enc, b_enc, W_dec, b_dec):
    raise NotImplementedError("write your pallas kernel here")


# speedup vs baseline: 32.5633x; 32.5633x over previous
import functools

import jax
import jax.numpy as jnp
from jax import lax
from jax.experimental import pallas as pl
from jax.experimental.pallas import tpu as pltpu

K_PER_ROW = 64
_POS_INF_BITS = 0x7F800000


def _encode_kernel(x_ref, w_ref, be_ref, bd_ref, out_ref):
    xb = x_ref[...] - bd_ref[...]
    acc = lax.dot_general(
        xb, w_ref[...], (((1,), (1,)), ((), ())),
        preferred_element_type=jnp.float32,
    )
    out_ref[...] = jnp.maximum(acc + be_ref[...], 0.0)


def _count_kernel(K_total, P, T, J, SHIFT, acts_ref, thr_ref, br_ref, cnt_ref):
    p = pl.program_id(0)
    t = pl.program_id(1)

    @pl.when((p == 0) & (t == 0))
    def _init():
        br_ref[0] = 0
        br_ref[1] = _POS_INF_BITS

    @pl.when(t == 0)
    def _zero():
        for j in range(J):
            cnt_ref[j] = 0

    lo = br_ref[0]
    hi = br_ref[1]
    step = jnp.maximum((hi - lo) >> SHIFT, 1)
    bits = lax.bitcast_convert_type(acts_ref[...], jnp.int32)
    for j in range(J):
        e = lo + (j + 1) * step
        cnt_ref[j] += jnp.sum((bits >= e).astype(jnp.int32))

    @pl.when(t == T - 1)
    def _update():
        lo_ = br_ref[0]
        hi_ = br_ref[1]
        step_ = jnp.maximum((hi_ - lo_) >> SHIFT, 1)
        new_lo = lo_
        new_hi = hi_
        for j in range(J):
            e = lo_ + (j + 1) * step_
            ge = cnt_ref[j] >= K_total
            new_lo = jnp.where(ge & (e > new_lo) & (e < hi_), e, new_lo)
            new_hi = jnp.where((~ge) & (e < new_hi), e, new_hi)
        br_ref[0] = new_lo
        br_ref[1] = new_hi

        @pl.when(p == P - 1)
        def _emit():
            thr_ref[0, 0] = new_lo


def _decode_kernel(thr_ref, acts_ref, w_ref, bd_ref, out_ref):
    k = pl.program_id(1)
    thr = thr_ref[0, 0]
    a = acts_ref[...]
    bits = lax.bitcast_convert_type(a, jnp.int32)
    enc = jnp.where(bits >= thr, a, 0.0)
    part = lax.dot_general(
        enc, w_ref[...], (((1,), (1,)), ((), ())),
        preferred_element_type=jnp.float32,
    )

    @pl.when(k == 0)
    def _first():
        out_ref[...] = part + bd_ref[...]

    @pl.when(k != 0)
    def _acc():
        out_ref[...] += part


def kernel(x, W_enc, b_enc, W_dec, b_dec):
    B, A = x.shape
    D = W_enc.shape[0]
    K_total = K_PER_ROW * B

    BT = min(512, B)
    DT = min(2048, D)
    acts = pl.pallas_call(
        _encode_kernel,
        grid=(D // DT, B // BT),
        in_specs=[
            pl.BlockSpec((BT, A), lambda j, i: (i, 0)),
            pl.BlockSpec((DT, A), lambda j, i: (j, 0)),
            pl.BlockSpec((1, DT), lambda j, i: (0, j)),
            pl.BlockSpec((1, A), lambda j, i: (0, 0)),
        ],
        out_specs=pl.BlockSpec((BT, DT), lambda j, i: (i, j)),
        out_shape=jax.ShapeDtypeStruct((B, D), jnp.float32),
    )(x, W_enc, b_enc.reshape(1, D), b_dec.reshape(1, A))

    RT = min(512, B)
    CT = min(4096, D)
    tr = B // RT
    tc = D // CT
    T = tr * tc
    P = 13
    SHIFT = 3
    J = (1 << SHIFT) - 1
    thr = pl.pallas_call(
        functools.partial(_count_kernel, K_total, P, T, J, SHIFT),
        grid=(P, T),
        in_specs=[pl.BlockSpec((RT, CT), lambda p, t: (t // tc, t % tc))],
        out_specs=pl.BlockSpec(memory_space=pltpu.SMEM),
        out_shape=jax.ShapeDtypeStruct((1, 1), jnp.int32),
        scratch_shapes=[
            pltpu.SMEM((2,), jnp.int32),
            pltpu.SMEM((J,), jnp.int32),
        ],
    )(acts)

    BT2 = min(1024, B)
    KT2 = min(2048, D)
    out = pl.pallas_call(
        _decode_kernel,
        grid=(B // BT2, D // KT2),
        in_specs=[
            pl.BlockSpec(memory_space=pltpu.SMEM),
            pl.BlockSpec((BT2, KT2), lambda i, k: (i, k)),
            pl.BlockSpec((A, KT2), lambda i, k: (0, k)),
            pl.BlockSpec((1, A), lambda i, k: (0, 0)),
        ],
        out_specs=pl.BlockSpec((BT2, A), lambda i, k: (i, 0)),
        out_shape=jax.ShapeDtypeStruct((B, A), jnp.float32),
    )(thr, acts, W_dec, b_dec.reshape(1, A))
    return out
